# Initial kernel scaffold; baseline (speedup 1.0000x reference)
import jax
import jax.numpy as jnp
from jax.experimental import pallas as pl


def kernel(x, anchors):
    raise NotImplementedError("write your pallas kernel here")


# speedup vs baseline: 1.3350x; 1.3350x over previous
import jax
import jax.numpy as jnp
from jax.experimental import pallas as pl

STRIDE = 8
A = 3
NC = 80
C = 5 + NC


def _decode_body(x_ref, m_ref, o_ref):
    v = x_ref[0]
    sig = jax.nn.sigmoid(v)
    ex = jnp.exp(v)
    row = jax.lax.broadcasted_iota(jnp.int32, v.shape, 0)
    col = jax.lax.broadcasted_iota(jnp.int32, v.shape, 1)
    use_exp = (row == 3) | (row == 4)
    base = jnp.where(use_exp, ex, sig)
    mult = m_ref[0]
    i_f = (col // 64).astype(jnp.float32)
    j_f = (col % 64).astype(jnp.float32)
    add = jnp.where(row == 1, jnp.float32(STRIDE) * i_f,
                    jnp.where(row == 2, jnp.float32(STRIDE) * j_f,
                              jnp.float32(0.0)))
    out = base * mult + add
    o_ref[0] = out.T


def kernel(x, anchors):
    B = x.shape[0]
    G = x.shape[2]
    P = G * G
    xs = x.reshape(B * A, C, P)

    ones = jnp.ones((A, C), dtype=jnp.float32)
    mult = ones.at[:, 1:3].set(jnp.float32(STRIDE))
    mult = mult.at[:, 3:5].set(anchors)
    mult = mult.reshape(A, C, 1)

    out = pl.pallas_call(
        _decode_body,
        grid=(B * A,),
        in_specs=[
            pl.BlockSpec((1, C, P), lambda g: (g, 0, 0)),
            pl.BlockSpec((1, C, 1), lambda g: (g % A, 0, 0)),
        ],
        out_specs=pl.BlockSpec((1, P, C), lambda g: (g, 0, 0)),
        out_shape=jax.ShapeDtypeStruct((B * A, P, C), jnp.float32),
    )(xs, mult)
    return out.reshape(B, A * P, C)


# speedup vs baseline: 1.5625x; 1.1704x over previous
import jax
import jax.numpy as jnp
from jax.experimental import pallas as pl

STRIDE = 8
A = 3
NC = 80
C = 5 + NC


def _decode_body(x_ref, m_ref, o_ref):
    v = x_ref[0]
    sig = jax.nn.sigmoid(v)
    ex = jnp.exp(v)
    ch = jax.lax.broadcasted_iota(jnp.int32, v.shape, 0)
    i_f = jax.lax.broadcasted_iota(jnp.int32, v.shape, 1).astype(jnp.float32)
    j_f = jax.lax.broadcasted_iota(jnp.int32, v.shape, 2).astype(jnp.float32)
    use_exp = (ch == 3) | (ch == 4)
    base = jnp.where(use_exp, ex, sig)
    mult = m_ref[0]
    add = jnp.where(ch == 1, jnp.float32(STRIDE) * i_f,
                    jnp.where(ch == 2, jnp.float32(STRIDE) * j_f,
                              jnp.float32(0.0)))
    dec = base * mult + add
    o_ref[0] = jnp.transpose(dec, (1, 2, 0))


def kernel(x, anchors):
    B = x.shape[0]
    G = x.shape[2]
    xs = x.reshape(B * A, C, G, G)

    ones = jnp.ones((A, C), dtype=jnp.float32)
    mult = ones.at[:, 1:3].set(jnp.float32(STRIDE))
    mult = mult.at[:, 3:5].set(anchors)
    mult = mult.reshape(A, C, 1, 1)

    out = pl.pallas_call(
        _decode_body,
        grid=(B * A,),
        in_specs=[
            pl.BlockSpec((1, C, G, G), lambda g: (g, 0, 0, 0)),
            pl.BlockSpec((1, C, 1, 1), lambda g: (g % A, 0, 0, 0)),
        ],
        out_specs=pl.BlockSpec((1, G, G, C), lambda g: (g, 0, 0, 0)),
        out_shape=jax.ShapeDtypeStruct((B * A, G, G, C), jnp.float32),
    )(xs, mult)
    return out.reshape(B, A * G * G, C)


# speedup vs baseline: 3.2361x; 2.0712x over previous
import jax
import jax.numpy as jnp
from jax.experimental import pallas as pl

STRIDE = 8
A = 3
NC = 80
C = 5 + NC

_BB = 8
_PL = 1024


def _decode_slab(v, mult, i_off):
    c_i = jax.lax.broadcasted_iota(jnp.int32, v.shape, 2)
    p_i = jax.lax.broadcasted_iota(jnp.int32, v.shape, 1)
    use_exp = (c_i == 3) | (c_i == 4)
    t = jnp.exp(jnp.where(use_exp, v, -v))
    base = jnp.where(use_exp, t, 1.0 / (1.0 + t))
    i_f = (p_i // 64 + i_off).astype(jnp.float32)
    j_f = (p_i % 64).astype(jnp.float32)
    add = jnp.where(c_i == 1, jnp.float32(STRIDE) * i_f,
                    jnp.where(c_i == 2, jnp.float32(STRIDE) * j_f,
                              jnp.float32(0.0)))
    return base * mult + add


def _body(x_ref, m_ref, o_ref):
    i_off = pl.program_id(1) * (_PL // 64)

    def run(lo):
        dec = _decode_slab(x_ref[:, :, lo:lo + C], m_ref[:, :, lo:lo + C],
                           i_off)
        for b in range(_BB):
            o_ref[:, b, :] = dec[b].T

    a = pl.program_id(2)
    for aa in range(A):
        @pl.when(a == aa)
        def _(lo=aa * C):
            run(lo)


def kernel(x, anchors):
    B = x.shape[0]
    G = x.shape[2]
    P = G * G
    xt = jnp.transpose(x, (0, 2, 3, 1)).reshape(B, P, A * C)

    ones = jnp.ones((A, C), dtype=jnp.float32)
    mult = ones.at[:, 1:3].set(jnp.float32(STRIDE))
    mult = mult.at[:, 3:5].set(anchors)
    mult = mult.reshape(1, 1, A * C)

    nb = B // _BB
    nl = P // _PL
    out = pl.pallas_call(
        _body,
        grid=(nb, nl, A),
        in_specs=[
            pl.BlockSpec((_BB, _PL, A * C), lambda b, l, a: (b, l, 0)),
            pl.BlockSpec((1, 1, A * C), lambda b, l, a: (0, 0, 0)),
        ],
        out_specs=pl.BlockSpec((C, _BB, _PL),
                               lambda b, l, a: (0, b, a * nl + l)),
        out_shape=jax.ShapeDtypeStruct((C, B, A * P), jnp.float32),
    )(xt, mult)
    return jnp.transpose(out, (1, 2, 0))


# speedup vs baseline: 3.5164x; 1.0866x over previous
import jax
import jax.numpy as jnp
from jax.experimental import pallas as pl

STRIDE = 8
A = 3
NC = 80
C = 5 + NC

_BB = 8
_PL = 1024


def _body(x_ref, t_ref, o_ref):
    i_off = pl.program_id(1) * (_PL // 64)

    def run(lo):
        sgn = t_ref[0, :, lo:lo + C]
        isexp = t_ref[1, :, lo:lo + C]
        notexp = t_ref[2, :, lo:lo + C]
        mult = t_ref[3, :, lo:lo + C]
        add_i = t_ref[4, :, lo:lo + C]
        add_j = t_ref[5, :, lo:lo + C]
        p_i = jax.lax.broadcasted_iota(jnp.int32, (_PL, 1), 0)
        i_f = (p_i // 64 + i_off).astype(jnp.float32)
        j_f = (p_i % 64).astype(jnp.float32)
        add_term = i_f * add_i + j_f * add_j
        for b in range(_BB):
            v = x_ref[b, :, lo:lo + C]
            t = jnp.exp(v * sgn)
            num = isexp * t + notexp
            den = notexp * t + 1.0
            dec = (num / den) * mult + add_term
            o_ref[:, b, :] = dec.T

    a = pl.program_id(2)
    for aa in range(A):
        @pl.when(a == aa)
        def _(lo=aa * C):
            run(lo)


def kernel(x, anchors):
    B = x.shape[0]
    G = x.shape[2]
    P = G * G
    xt = jnp.transpose(x, (0, 2, 3, 1)).reshape(B, P, A * C)

    f32 = jnp.float32
    isexp_row = jnp.zeros((A, C), f32).at[:, 3:5].set(1.0)
    sgn_row = 2.0 * isexp_row - 1.0
    mult_row = jnp.ones((A, C), f32).at[:, 1:3].set(f32(STRIDE))
    mult_row = mult_row.at[:, 3:5].set(anchors)
    addi_row = jnp.zeros((A, C), f32).at[:, 1].set(f32(STRIDE))
    addj_row = jnp.zeros((A, C), f32).at[:, 2].set(f32(STRIDE))
    tab = jnp.stack([sgn_row, isexp_row, 1.0 - isexp_row, mult_row,
                     addi_row, addj_row]).reshape(6, 1, A * C)

    nb = B // _BB
    nl = P // _PL
    out = pl.pallas_call(
        _body,
        grid=(nb, nl, A),
        in_specs=[
            pl.BlockSpec((_BB, _PL, A * C), lambda b, l, a: (b, l, 0)),
            pl.BlockSpec((6, 1, A * C), lambda b, l, a: (0, 0, 0)),
        ],
        out_specs=pl.BlockSpec((C, _BB, _PL),
                               lambda b, l, a: (0, b, a * nl + l)),
        out_shape=jax.ShapeDtypeStruct((C, B, A * P), jnp.float32),
    )(xt, tab)
    return jnp.transpose(out, (1, 2, 0))


# speedup vs baseline: 3.8320x; 1.0897x over previous
import jax
import jax.numpy as jnp
from jax.experimental import pallas as pl
from jax.experimental.pallas import tpu as pltpu

STRIDE = 8
A = 3
NC = 80
C = 5 + NC

_BB = 8
_PL = 1024


def _body(nb, nl, x_hbm, t_ref, o_ref, buf, sem):
    bi = pl.program_id(0)
    li = pl.program_id(1)
    a = pl.program_id(2)
    k = bi * nl + li
    slot = jax.lax.rem(k, 2)
    i_off = li * (_PL // 64)

    def fetch(kk, ss):
        b2 = jax.lax.div(kk, nl)
        l2 = jax.lax.rem(kk, nl)
        pltpu.make_async_copy(
            x_hbm.at[pl.ds(b2 * _BB, _BB), pl.ds(l2 * _PL, _PL), :],
            buf.at[ss],
            sem.at[ss],
        ).start()

    @pl.when(a == 0)
    def _():
        @pl.when(k == 0)
        def _():
            fetch(0, 0)

        @pl.when(k + 1 < nb * nl)
        def _():
            fetch(k + 1, 1 - slot)

        pltpu.make_async_copy(
            x_hbm.at[pl.ds(bi * _BB, _BB), pl.ds(li * _PL, _PL), :],
            buf.at[slot],
            sem.at[slot],
        ).wait()

    x_ref = buf.at[slot]

    def run(lo):
        sgn = t_ref[0, :, lo:lo + C]
        isexp = t_ref[1, :, lo:lo + C]
        notexp = t_ref[2, :, lo:lo + C]
        mult = t_ref[3, :, lo:lo + C]
        add_i = t_ref[4, :, lo:lo + C]
        add_j = t_ref[5, :, lo:lo + C]
        p_i = jax.lax.broadcasted_iota(jnp.int32, (_PL, 1), 0)
        i_f = (p_i // 64 + i_off).astype(jnp.float32)
        j_f = (p_i % 64).astype(jnp.float32)
        add_term = i_f * add_i + j_f * add_j
        for b in range(_BB):
            v = x_ref[b, :, lo:lo + C]
            t = jnp.exp(v * sgn)
            num = isexp * t + notexp
            den = notexp * t + 1.0
            dec = (num / den) * mult + add_term
            o_ref[:, b, :] = dec.T

    for aa in range(A):
        @pl.when(a == aa)
        def _(lo=aa * C):
            run(lo)


def kernel(x, anchors):
    B = x.shape[0]
    G = x.shape[2]
    P = G * G
    xt = jnp.transpose(x, (0, 2, 3, 1)).reshape(B, P, A * C)

    f32 = jnp.float32
    isexp_row = jnp.zeros((A, C), f32).at[:, 3:5].set(1.0)
    sgn_row = 2.0 * isexp_row - 1.0
    mult_row = jnp.ones((A, C), f32).at[:, 1:3].set(f32(STRIDE))
    mult_row = mult_row.at[:, 3:5].set(anchors)
    addi_row = jnp.zeros((A, C), f32).at[:, 1].set(f32(STRIDE))
    addj_row = jnp.zeros((A, C), f32).at[:, 2].set(f32(STRIDE))
    tab = jnp.stack([sgn_row, isexp_row, 1.0 - isexp_row, mult_row,
                     addi_row, addj_row]).reshape(6, 1, A * C)

    nb = B // _BB
    nl = P // _PL
    import functools
    out = pl.pallas_call(
        functools.partial(_body, nb, nl),
        grid=(nb, nl, A),
        in_specs=[
            pl.BlockSpec(memory_space=pl.ANY),
            pl.BlockSpec((6, 1, A * C), lambda b, l, a: (0, 0, 0)),
        ],
        out_specs=pl.BlockSpec((C, _BB, _PL),
                               lambda b, l, a: (0, b, a * nl + l)),
        out_shape=jax.ShapeDtypeStruct((C, B, A * P), jnp.float32),
        scratch_shapes=[
            pltpu.VMEM((2, _BB, _PL, A * C), jnp.float32),
            pltpu.SemaphoreType.DMA((2,)),
        ],
    )(xt, tab)
    return jnp.transpose(out, (1, 2, 0))


# speedup vs baseline: 4.2038x; 1.0970x over previous
import jax
import jax.numpy as jnp
from jax.experimental import pallas as pl
from jax.experimental.pallas import tpu as pltpu

STRIDE = 8
A = 3
NC = 80
C = 5 + NC

_BB = 8
_PL = 1024


def _body(nb, nl, x_hbm, t_ref, o_ref, buf, sem):
    bi = pl.program_id(0)
    li = pl.program_id(1)
    a = pl.program_id(2)
    k = bi * nl + li
    slot = jax.lax.rem(k, 2)
    i_off = li * (_PL // 64)

    def fetch(kk, ss):
        b2 = jax.lax.div(kk, nl)
        l2 = jax.lax.rem(kk, nl)
        pltpu.make_async_copy(
            x_hbm.at[pl.ds(b2 * _BB, _BB), pl.ds(l2 * _PL, _PL), :],
            buf.at[ss],
            sem.at[ss],
        ).start()

    @pl.when(a == 0)
    def _():
        @pl.when(k == 0)
        def _():
            fetch(0, 0)

        @pl.when(k + 1 < nb * nl)
        def _():
            fetch(k + 1, 1 - slot)

        pltpu.make_async_copy(
            x_hbm.at[pl.ds(bi * _BB, _BB), pl.ds(li * _PL, _PL), :],
            buf.at[slot],
            sem.at[slot],
        ).wait()

    x_ref = buf.at[slot]

    def run(lo):
        sgn = t_ref[0, :, lo:lo + C]
        isexp = t_ref[1, :, lo:lo + C]
        invm = t_ref[2, :, lo:lo + C]
        em = t_ref[3, :, lo:lo + C]
        add_i = t_ref[4, :, lo:lo + C]
        add_j = t_ref[5, :, lo:lo + C]
        mask = isexp != 0.0
        p_i = jax.lax.broadcasted_iota(jnp.int32, (_PL, 1), 0)
        i_f = (p_i // 64 + i_off).astype(jnp.float32)
        j_f = (p_i % 64).astype(jnp.float32)
        add_term = i_f * add_i + j_f * add_j
        for b in range(_BB):
            v = x_ref[b, :, lo:lo + C]
            t = jnp.exp(v * sgn)
            den = t * invm + invm
            dec = jnp.where(mask, t * em, 1.0 / den) + add_term
            o_ref[:, b, :] = dec.T

    for aa in range(A):
        @pl.when(a == aa)
        def _(lo=aa * C):
            run(lo)


def kernel(x, anchors):
    B = x.shape[0]
    G = x.shape[2]
    P = G * G
    xt = jnp.transpose(x, (0, 2, 3, 1)).reshape(B, P, A * C)

    f32 = jnp.float32
    isexp_row = jnp.zeros((A, C), f32).at[:, 3:5].set(1.0)
    sgn_row = 2.0 * isexp_row - 1.0
    invm_row = jnp.ones((A, C), f32).at[:, 1:3].set(f32(1.0 / STRIDE))
    invm_row = invm_row.at[:, 3:5].set(1.0)
    em_row = jnp.zeros((A, C), f32).at[:, 3:5].set(anchors)
    addi_row = jnp.zeros((A, C), f32).at[:, 1].set(f32(STRIDE))
    addj_row = jnp.zeros((A, C), f32).at[:, 2].set(f32(STRIDE))
    tab = jnp.stack([sgn_row, isexp_row, invm_row, em_row,
                     addi_row, addj_row]).reshape(6, 1, A * C)

    nb = B // _BB
    nl = P // _PL
    import functools
    out = pl.pallas_call(
        functools.partial(_body, nb, nl),
        grid=(nb, nl, A),
        in_specs=[
            pl.BlockSpec(memory_space=pl.ANY),
            pl.BlockSpec((6, 1, A * C), lambda b, l, a: (0, 0, 0)),
        ],
        out_specs=pl.BlockSpec((C, _BB, _PL),
                               lambda b, l, a: (0, b, a * nl + l)),
        out_shape=jax.ShapeDtypeStruct((C, B, A * P), jnp.float32),
        scratch_shapes=[
            pltpu.VMEM((2, _BB, _PL, A * C), jnp.float32),
            pltpu.SemaphoreType.DMA((2,)),
        ],
    )(xt, tab)
    return jnp.transpose(out, (1, 2, 0))


# speedup vs baseline: 4.3221x; 1.0281x over previous
import jax
import jax.numpy as jnp
from jax.experimental import pallas as pl
from jax.experimental.pallas import tpu as pltpu

STRIDE = 8
A = 3
NC = 80
C = 5 + NC

_BB = 8
_PL = 1024


def _body(nb, nl, x_hbm, t_ref, o_ref, buf, add_buf, sem):
    bi = pl.program_id(0)
    li = pl.program_id(1)
    a = pl.program_id(2)
    k = bi * nl + li
    slot = jax.lax.rem(k, 2)
    i_off = li * (_PL // 64)

    def fetch(kk, ss):
        b2 = jax.lax.div(kk, nl)
        l2 = jax.lax.rem(kk, nl)
        pltpu.make_async_copy(
            x_hbm.at[pl.ds(b2 * _BB, _BB), pl.ds(l2 * _PL, _PL), :],
            buf.at[ss],
            sem.at[ss],
        ).start()

    @pl.when(a == 0)
    def _():
        @pl.when(k == 0)
        def _():
            fetch(0, 0)

        @pl.when(k + 1 < nb * nl)
        def _():
            fetch(k + 1, 1 - slot)

        add_i = t_ref[4, :, 0:C]
        add_j = t_ref[5, :, 0:C]
        p_i = jax.lax.broadcasted_iota(jnp.int32, (_PL, 1), 0)
        i_f = (p_i // 64 + i_off).astype(jnp.float32)
        j_f = (p_i % 64).astype(jnp.float32)
        add_buf[...] = i_f * add_i + j_f * add_j

        pltpu.make_async_copy(
            x_hbm.at[pl.ds(bi * _BB, _BB), pl.ds(li * _PL, _PL), :],
            buf.at[slot],
            sem.at[slot],
        ).wait()

    x_ref = buf.at[slot]

    def run(lo):
        sgn = t_ref[0, :, lo:lo + C]
        isexp = t_ref[1, :, lo:lo + C]
        invm = t_ref[2, :, lo:lo + C]
        em = t_ref[3, :, lo:lo + C]
        mask = isexp != 0.0
        add_term = add_buf[...]
        for b in range(_BB):
            v = x_ref[b, :, lo:lo + C]
            t = jnp.exp(v * sgn)
            den = t * invm + invm
            dec = jnp.where(mask, t * em, 1.0 / den) + add_term
            o_ref[:, b, :] = dec.T

    for aa in range(A):
        @pl.when(a == aa)
        def _(lo=aa * C):
            run(lo)


def kernel(x, anchors):
    B = x.shape[0]
    G = x.shape[2]
    P = G * G
    xt = jnp.transpose(x, (0, 2, 3, 1)).reshape(B, P, A * C)

    f32 = jnp.float32
    isexp_row = jnp.zeros((A, C), f32).at[:, 3:5].set(1.0)
    sgn_row = 2.0 * isexp_row - 1.0
    invm_row = jnp.ones((A, C), f32).at[:, 1:3].set(f32(1.0 / STRIDE))
    invm_row = invm_row.at[:, 3:5].set(1.0)
    em_row = jnp.zeros((A, C), f32).at[:, 3:5].set(anchors)
    addi_row = jnp.zeros((A, C), f32).at[:, 1].set(f32(STRIDE))
    addj_row = jnp.zeros((A, C), f32).at[:, 2].set(f32(STRIDE))
    tab = jnp.stack([sgn_row, isexp_row, invm_row, em_row,
                     addi_row, addj_row]).reshape(6, 1, A * C)

    nb = B // _BB
    nl = P // _PL
    import functools
    out = pl.pallas_call(
        functools.partial(_body, nb, nl),
        grid=(nb, nl, A),
        in_specs=[
            pl.BlockSpec(memory_space=pl.ANY),
            pl.BlockSpec((6, 1, A * C), lambda b, l, a: (0, 0, 0)),
        ],
        out_specs=pl.BlockSpec((C, _BB, _PL),
                               lambda b, l, a: (0, b, a * nl + l)),
        out_shape=jax.ShapeDtypeStruct((C, B, A * P), jnp.float32),
        scratch_shapes=[
            pltpu.VMEM((2, _BB, _PL, A * C), jnp.float32),
            pltpu.VMEM((_PL, C), jnp.float32),
            pltpu.SemaphoreType.DMA((2,)),
        ],
    )(xt, tab)
    return jnp.transpose(out, (1, 2, 0))


# speedup vs baseline: 4.4655x; 1.0332x over previous
import jax
import jax.numpy as jnp
from jax.experimental import pallas as pl
from jax.experimental.pallas import tpu as pltpu

STRIDE = 8
A = 3
NC = 80
C = 5 + NC

_BB = 8
_PL = 512


def _body(nb, nl, x_hbm, t_ref, o_ref, buf, add_buf, sem):
    bi = pl.program_id(0)
    li = pl.program_id(1)
    a = pl.program_id(2)
    k = bi * nl + li
    slot = jax.lax.rem(k, 2)
    i_off = li * (_PL // 64)

    def fetch(kk, ss):
        b2 = jax.lax.div(kk, nl)
        l2 = jax.lax.rem(kk, nl)
        pltpu.make_async_copy(
            x_hbm.at[pl.ds(b2 * _BB, _BB), pl.ds(l2 * _PL, _PL), :],
            buf.at[ss],
            sem.at[ss],
        ).start()

    @pl.when(a == 0)
    def _():
        @pl.when(k == 0)
        def _():
            fetch(0, 0)

        @pl.when(k + 1 < nb * nl)
        def _():
            fetch(k + 1, 1 - slot)

        add_i = t_ref[4, :, 0:C]
        add_j = t_ref[5, :, 0:C]
        p_i = jax.lax.broadcasted_iota(jnp.int32, (_PL, 1), 0)
        i_f = (p_i // 64 + i_off).astype(jnp.float32)
        j_f = (p_i % 64).astype(jnp.float32)
        add_buf[...] = i_f * add_i + j_f * add_j

        pltpu.make_async_copy(
            x_hbm.at[pl.ds(bi * _BB, _BB), pl.ds(li * _PL, _PL), :],
            buf.at[slot],
            sem.at[slot],
        ).wait()

    x_ref = buf.at[slot]

    def run(lo):
        sgn = t_ref[0, :, lo:lo + C]
        isexp = t_ref[1, :, lo:lo + C]
        invm = t_ref[2, :, lo:lo + C]
        em = t_ref[3, :, lo:lo + C]
        mask = isexp != 0.0
        add_term = add_buf[...]
        for b in range(_BB):
            v = x_ref[b, :, lo:lo + C]
            t = jnp.exp(v * sgn)
            den = t * invm + invm
            dec = jnp.where(mask, t * em, 1.0 / den) + add_term
            o_ref[:, b, :] = dec.T

    for aa in range(A):
        @pl.when(a == aa)
        def _(lo=aa * C):
            run(lo)


def kernel(x, anchors):
    B = x.shape[0]
    G = x.shape[2]
    P = G * G
    xt = jnp.transpose(x, (0, 2, 3, 1)).reshape(B, P, A * C)

    f32 = jnp.float32
    isexp_row = jnp.zeros((A, C), f32).at[:, 3:5].set(1.0)
    sgn_row = 2.0 * isexp_row - 1.0
    invm_row = jnp.ones((A, C), f32).at[:, 1:3].set(f32(1.0 / STRIDE))
    invm_row = invm_row.at[:, 3:5].set(1.0)
    em_row = jnp.zeros((A, C), f32).at[:, 3:5].set(anchors)
    addi_row = jnp.zeros((A, C), f32).at[:, 1].set(f32(STRIDE))
    addj_row = jnp.zeros((A, C), f32).at[:, 2].set(f32(STRIDE))
    tab = jnp.stack([sgn_row, isexp_row, invm_row, em_row,
                     addi_row, addj_row]).reshape(6, 1, A * C)

    nb = B // _BB
    nl = P // _PL
    import functools
    out = pl.pallas_call(
        functools.partial(_body, nb, nl),
        grid=(nb, nl, A),
        in_specs=[
            pl.BlockSpec(memory_space=pl.ANY),
            pl.BlockSpec((6, 1, A * C), lambda b, l, a: (0, 0, 0)),
        ],
        out_specs=pl.BlockSpec((C, _BB, _PL),
                               lambda b, l, a: (0, b, a * nl + l)),
        out_shape=jax.ShapeDtypeStruct((C, B, A * P), jnp.float32),
        scratch_shapes=[
            pltpu.VMEM((2, _BB, _PL, A * C), jnp.float32),
            pltpu.VMEM((_PL, C), jnp.float32),
            pltpu.SemaphoreType.DMA((2,)),
        ],
    )(xt, tab)
    return jnp.transpose(out, (1, 2, 0))
